# baseline (speedup 1.0000x reference)
import jax
import jax.numpy as jnp
from jax import lax
from jax.experimental import pallas as pl
from jax.experimental.pallas import tpu as pltpu
from jax.experimental.pallas import tpu_sc as plsc

N = 10000
D = 256
H = 64
W = 2 * H
E = 160000

NUM_TILES = 32
CHUNK = 128
K = 40
E_PAD = NUM_TILES * K * CHUNK
ACC_ROWS = 10240
ZERO_ROWS_PER_TILE = ACC_ROWS // 16

BLK = 1000
GRID = N // BLK


NBUF = 2


def _sc_scatter_body(p_hbm, src_hbm, dst_hbm, zeros_hbm, out_hbm,
                     src_v, dst_v, buf0, buf1,
                     acc, g0, g1, s0, s1):
    c = lax.axis_index("c")
    s = lax.axis_index("s")
    tid = c * 16 + s
    bufs = (buf0, buf1)
    gsem = (g0, g1)
    ssem = (s0, s1)

    pltpu.sync_copy(src_hbm.at[tid], src_v)
    pltpu.sync_copy(dst_hbm.at[tid], dst_v)
    for b in range(NBUF):
        pltpu.async_copy(p_hbm.at[src_v.at[b]], bufs[b], gsem[b])

    pltpu.sync_copy(zeros_hbm.at[pl.ds(s * ZERO_ROWS_PER_TILE, ZERO_ROWS_PER_TILE)],
                    acc.at[pl.ds(s * ZERO_ROWS_PER_TILE, ZERO_ROWS_PER_TILE)])
    plsc.subcore_barrier()

    def body(i, _):
        js = [i * NBUF + b for b in range(NBUF)]
        for b in range(NBUF):
            pltpu.make_async_copy(p_hbm.at[src_v.at[js[b]]], bufs[b],
                                  gsem[b]).wait()
            pltpu.async_copy(bufs[b], acc.at[dst_v.at[js[b]]], ssem[b],
                             add=True)
        for b in range(NBUF):
            @pl.when(js[b] + NBUF < K)
            def _():
                pltpu.make_async_copy(bufs[b], acc.at[dst_v.at[js[b]]],
                                      ssem[b]).wait()
                pltpu.async_copy(p_hbm.at[src_v.at[js[b] + NBUF]], bufs[b],
                                 gsem[b])
        return 0

    lax.fori_loop(0, K // NBUF, body, 0)
    for b in range(NBUF):
        pltpu.make_async_copy(bufs[b], acc.at[dst_v.at[K - NBUF + b]],
                              ssem[b]).wait()
    plsc.subcore_barrier()

    pltpu.sync_copy(acc.at[pl.ds(s * ZERO_ROWS_PER_TILE, ZERO_ROWS_PER_TILE)],
                    out_hbm.at[c, pl.ds(s * ZERO_ROWS_PER_TILE, ZERO_ROWS_PER_TILE)])


@jax.jit
def _sc_scatter(p, src_t, dst_t, zeros_t):
    mesh = plsc.VectorSubcoreMesh(core_axis_name="c", subcore_axis_name="s")
    return pl.kernel(
        _sc_scatter_body,
        out_type=jax.ShapeDtypeStruct((2, ACC_ROWS, W), jnp.float32),
        mesh=mesh,
        scratch_types=(
            [pltpu.VMEM((K, CHUNK), jnp.int32)] * 2
            + [pltpu.VMEM((CHUNK, W), jnp.float32)] * NBUF
            + [pltpu.VMEM_SHARED((ACC_ROWS, W), jnp.float32)]
            + [pltpu.SemaphoreType.DMA] * (2 * NBUF)
        ),
    )(p, src_t, dst_t, zeros_t)


def _proj1_body(x_ref, wcat_ref, rb_ref, y_ref):
    y = jnp.dot(x_ref[...], wcat_ref[...], preferred_element_type=jnp.float32, precision=lax.Precision.HIGHEST)
    y_ref[...] = jnp.concatenate(
        [y[:, :H], jnp.maximum(y[:, H:] + rb_ref[...], 0.0)], axis=1)


@jax.jit
def _proj1(x, wcat, rb):
    return pl.pallas_call(
        _proj1_body,
        grid=(GRID,),
        in_specs=[
            pl.BlockSpec((BLK, D), lambda i: (i, 0)),
            pl.BlockSpec((D, W), lambda i: (0, 0)),
            pl.BlockSpec((1, H), lambda i: (0, 0)),
        ],
        out_specs=pl.BlockSpec((BLK, W), lambda i: (i, 0)),
        out_shape=jax.ShapeDtypeStruct((N, W), jnp.float32),
    )(x, wcat, rb)


def _bn(acc_ref, y_ref, b_ref, g_ref, be_ref, m_ref, v_ref):
    agg = acc_ref[0, :, :H] + acc_ref[1, :, :H]
    out = jnp.maximum(agg + b_ref[...], 0.0) + y_ref[:, H:]
    scale = g_ref[...] * lax.rsqrt(v_ref[...] + 1e-5)
    return (out - m_ref[...]) * scale + be_ref[...]


def _epi1_body(acc_ref, y1_ref, b_ref, g_ref, be_ref, m_ref, v_ref,
               wcat_ref, rb2_ref, y2_ref):
    h1 = _bn(acc_ref, y1_ref, b_ref, g_ref, be_ref, m_ref, v_ref)
    y = jnp.dot(h1, wcat_ref[...], preferred_element_type=jnp.float32, precision=lax.Precision.HIGHEST)
    y2_ref[...] = jnp.concatenate(
        [y[:, :H], jnp.maximum(y[:, H:] + rb2_ref[...], 0.0)], axis=1)


@jax.jit
def _epi1(acc, y1, b, g, be, m, v, wcat2, rb2):
    vec = pl.BlockSpec((1, H), lambda i: (0, 0))
    return pl.pallas_call(
        _epi1_body,
        grid=(GRID,),
        in_specs=[
            pl.BlockSpec((2, BLK, W), lambda i: (0, i, 0)),
            pl.BlockSpec((BLK, W), lambda i: (i, 0)),
            vec, vec, vec, vec, vec,
            pl.BlockSpec((H, W), lambda i: (0, 0)),
            vec,
        ],
        out_specs=pl.BlockSpec((BLK, W), lambda i: (i, 0)),
        out_shape=jax.ShapeDtypeStruct((N, W), jnp.float32),
    )(acc, y1, b, g, be, m, v, wcat2, rb2)


def _final_body(acc_ref, y2_ref, b_ref, g_ref, be_ref, m_ref, v_ref,
                aw_ref, ab_ref, pw1_ref, pb1_ref, pw2_ref, pb2_ref,
                vw1_ref, vb1_ref, vw2_ref, vb2_ref,
                pol_ref, val_ref, wsum_ref, hmax_ref):
    i = pl.program_id(0)
    h2 = _bn(acc_ref, y2_ref, b_ref, g_ref, be_ref, m_ref, v_ref)
    logits = jnp.sum(h2 * aw_ref[...], axis=1, keepdims=True) + ab_ref[...]
    w = 1.0 / (1.0 + jnp.exp(-logits))
    part_sum = jnp.sum(w * h2, axis=0, keepdims=True)
    part_max = jnp.max(h2, axis=0, keepdims=True)

    @pl.when(i == 0)
    def _():
        wsum_ref[...] = part_sum
        hmax_ref[...] = part_max

    @pl.when(i > 0)
    def _():
        wsum_ref[...] = wsum_ref[...] + part_sum
        hmax_ref[...] = jnp.maximum(hmax_ref[...], part_max)

    @pl.when(i == GRID - 1)
    def _():
        gf = jnp.concatenate([wsum_ref[...], hmax_ref[...]], axis=1)
        p_hidden = jnp.maximum(
            jnp.dot(gf, pw1_ref[...], preferred_element_type=jnp.float32, precision=lax.Precision.HIGHEST)
            + pb1_ref[...], 0.0)
        pol_ref[...] = (jnp.dot(p_hidden, pw2_ref[...],
                                preferred_element_type=jnp.float32, precision=lax.Precision.HIGHEST)
                        + pb2_ref[...])
        v_hidden = jnp.maximum(
            jnp.dot(gf, vw1_ref[...], preferred_element_type=jnp.float32, precision=lax.Precision.HIGHEST)
            + vb1_ref[...], 0.0)
        val_ref[...] = jnp.tanh(
            jnp.sum(v_hidden * vw2_ref[...], axis=1, keepdims=True)
            + vb2_ref[...])


@jax.jit
def _final(acc, y2, b, g, be, m, v, aw_row, ab, pw1, pb1, pw2, pb2,
           vw1, vb1, vw2_row, vb2):
    vec = pl.BlockSpec((1, H), lambda i: (0, 0))
    one = pl.BlockSpec((1, 1), lambda i: (0, 0))
    return pl.pallas_call(
        _final_body,
        grid=(GRID,),
        in_specs=[
            pl.BlockSpec((2, BLK, W), lambda i: (0, i, 0)),
            pl.BlockSpec((BLK, W), lambda i: (i, 0)),
            vec, vec, vec, vec, vec,
            vec, one,
            pl.BlockSpec((W, 128), lambda i: (0, 0)),
            pl.BlockSpec((1, 128), lambda i: (0, 0)),
            pl.BlockSpec((128, 128), lambda i: (0, 0)),
            pl.BlockSpec((1, 128), lambda i: (0, 0)),
            pl.BlockSpec((W, H), lambda i: (0, 0)),
            vec, vec, one,
        ],
        out_specs=[
            pl.BlockSpec((1, 128), lambda i: (0, 0)),
            pl.BlockSpec((1, 1), lambda i: (0, 0)),
        ],
        out_shape=[
            jax.ShapeDtypeStruct((1, 128), jnp.float32),
            jax.ShapeDtypeStruct((1, 1), jnp.float32),
        ],
        scratch_shapes=[
            pltpu.VMEM((1, H), jnp.float32),
            pltpu.VMEM((1, H), jnp.float32),
        ],
    )(acc, y2, b, g, be, m, v, aw_row, ab, pw1, pb1, pw2, pb2,
      vw1, vb1, vw2_row, vb2)


def kernel(x, edge_index, W1, b1, rW1, rb1, g1, be1, m1, v1,
           W2, b2, rW2, rb2, g2, be2, m2, v2, aW, ab,
           pW1, pb1, pW2, pb2, vW1, vb1, vW2, vb2):
    src = edge_index[0]
    dst = edge_index[1]
    pad = E_PAD - E
    pad_iota = jnp.arange(pad, dtype=jnp.int32)
    src_t = jnp.concatenate([src, pad_iota % N]
                            ).reshape(NUM_TILES, K, CHUNK)
    dst_t = jnp.concatenate([dst, N + pad_iota % (ACC_ROWS - N)]
                            ).reshape(NUM_TILES, K, CHUNK)
    zeros_t = jnp.zeros((ACC_ROWS, W), jnp.float32)

    wcat1 = jnp.concatenate([W1, rW1], axis=1)
    wcat2 = jnp.concatenate([W2, rW2], axis=1)
    row = lambda a: a.reshape(1, -1)

    y1 = _proj1(x, wcat1, row(rb1))
    acc1 = _sc_scatter(y1, src_t, dst_t, zeros_t)
    y2 = _epi1(acc1, y1, row(b1), row(g1), row(be1), row(m1), row(v1),
               wcat2, row(rb2))
    acc2 = _sc_scatter(y2, src_t, dst_t, zeros_t)
    pol, val = _final(acc2, y2, row(b2), row(g2), row(be2), row(m2), row(v2),
                      row(aW.T), row(ab), pW1, row(pb1), pW2, row(pb2),
                      vW1, row(vb1), row(vW2.T), row(vb2))
    return (pol, val)


# speedup vs baseline: 1.0316x; 1.0316x over previous
import jax
import jax.numpy as jnp
from jax import lax
from jax.experimental import pallas as pl
from jax.experimental.pallas import tpu as pltpu
from jax.experimental.pallas import tpu_sc as plsc

N = 10000
D = 256
H = 64
W = 2 * H
E = 160000

NUM_TILES = 32
CHUNK = 128
K = 40
E_PAD = NUM_TILES * K * CHUNK
ACC_ROWS = 10240
ZERO_ROWS_PER_TILE = ACC_ROWS // 16

BLK = 1000
GRID = N // BLK


NBUF = 2


def _sc_scatter_body(p_hbm, src_hbm, dst_hbm, zeros_hbm, out_hbm,
                     src_v, dst_v, buf0, buf1,
                     acc, g0, g1, s0, s1):
    c = lax.axis_index("c")
    s = lax.axis_index("s")
    tid = c * 16 + s
    bufs = (buf0, buf1)
    gsem = (g0, g1)
    ssem = (s0, s1)

    pltpu.sync_copy(src_hbm.at[tid], src_v)
    pltpu.sync_copy(dst_hbm.at[tid], dst_v)
    for b in range(NBUF):
        pltpu.async_copy(p_hbm.at[src_v.at[b]], bufs[b], gsem[b])

    pltpu.sync_copy(zeros_hbm.at[pl.ds(s * ZERO_ROWS_PER_TILE, ZERO_ROWS_PER_TILE)],
                    acc.at[pl.ds(s * ZERO_ROWS_PER_TILE, ZERO_ROWS_PER_TILE)])
    plsc.subcore_barrier()

    def body(i, _):
        js = [i * NBUF + b for b in range(NBUF)]
        for b in range(NBUF):
            pltpu.make_async_copy(p_hbm.at[src_v.at[js[b]]], bufs[b],
                                  gsem[b]).wait()
            pltpu.async_copy(bufs[b], acc.at[dst_v.at[js[b]]], ssem[b],
                             add=True)
        for b in range(NBUF):
            @pl.when(js[b] + NBUF < K)
            def _():
                pltpu.make_async_copy(bufs[b], acc.at[dst_v.at[js[b]]],
                                      ssem[b]).wait()
                pltpu.async_copy(p_hbm.at[src_v.at[js[b] + NBUF]], bufs[b],
                                 gsem[b])
        return 0

    lax.fori_loop(0, K // NBUF, body, 0)
    for b in range(NBUF):
        pltpu.make_async_copy(bufs[b], acc.at[dst_v.at[K - NBUF + b]],
                              ssem[b]).wait()
    plsc.subcore_barrier()

    pltpu.sync_copy(acc.at[pl.ds(s * ZERO_ROWS_PER_TILE, ZERO_ROWS_PER_TILE)],
                    out_hbm.at[c, pl.ds(s * ZERO_ROWS_PER_TILE, ZERO_ROWS_PER_TILE)])


@jax.jit
def _sc_scatter(p, src_t, dst_t, zeros_t):
    mesh = plsc.VectorSubcoreMesh(core_axis_name="c", subcore_axis_name="s")
    return pl.kernel(
        _sc_scatter_body,
        out_type=jax.ShapeDtypeStruct((2, ACC_ROWS, W), jnp.float32),
        mesh=mesh,
        scratch_types=(
            [pltpu.VMEM((K, CHUNK), jnp.int32)] * 2
            + [pltpu.VMEM((CHUNK, W), jnp.float32)] * NBUF
            + [pltpu.VMEM_SHARED((ACC_ROWS, W), jnp.float32)]
            + [pltpu.SemaphoreType.DMA] * (2 * NBUF)
        ),
    )(p, src_t, dst_t, zeros_t)


def _proj1_body(x_ref, wcat_ref, rb_ref, y_ref):
    y = jnp.dot(x_ref[...], wcat_ref[...], preferred_element_type=jnp.float32)
    y_ref[...] = jnp.concatenate(
        [y[:, :H], jnp.maximum(y[:, H:] + rb_ref[...], 0.0)], axis=1)


@jax.jit
def _proj1(x, wcat, rb):
    return pl.pallas_call(
        _proj1_body,
        grid=(GRID,),
        in_specs=[
            pl.BlockSpec((BLK, D), lambda i: (i, 0)),
            pl.BlockSpec((D, W), lambda i: (0, 0)),
            pl.BlockSpec((1, H), lambda i: (0, 0)),
        ],
        out_specs=pl.BlockSpec((BLK, W), lambda i: (i, 0)),
        out_shape=jax.ShapeDtypeStruct((N, W), jnp.float32),
    )(x, wcat, rb)


def _bn(acc_ref, y_ref, b_ref, g_ref, be_ref, m_ref, v_ref):
    agg = acc_ref[0, :, :H] + acc_ref[1, :, :H]
    out = jnp.maximum(agg + b_ref[...], 0.0) + y_ref[:, H:]
    scale = g_ref[...] * lax.rsqrt(v_ref[...] + 1e-5)
    return (out - m_ref[...]) * scale + be_ref[...]


def _epi1_body(acc_ref, y1_ref, b_ref, g_ref, be_ref, m_ref, v_ref,
               wcat_ref, rb2_ref, y2_ref):
    h1 = _bn(acc_ref, y1_ref, b_ref, g_ref, be_ref, m_ref, v_ref)
    y = jnp.dot(h1, wcat_ref[...], preferred_element_type=jnp.float32)
    y2_ref[...] = jnp.concatenate(
        [y[:, :H], jnp.maximum(y[:, H:] + rb2_ref[...], 0.0)], axis=1)


@jax.jit
def _epi1(acc, y1, b, g, be, m, v, wcat2, rb2):
    vec = pl.BlockSpec((1, H), lambda i: (0, 0))
    return pl.pallas_call(
        _epi1_body,
        grid=(GRID,),
        in_specs=[
            pl.BlockSpec((2, BLK, W), lambda i: (0, i, 0)),
            pl.BlockSpec((BLK, W), lambda i: (i, 0)),
            vec, vec, vec, vec, vec,
            pl.BlockSpec((H, W), lambda i: (0, 0)),
            vec,
        ],
        out_specs=pl.BlockSpec((BLK, W), lambda i: (i, 0)),
        out_shape=jax.ShapeDtypeStruct((N, W), jnp.float32),
    )(acc, y1, b, g, be, m, v, wcat2, rb2)


def _final_body(acc_ref, y2_ref, b_ref, g_ref, be_ref, m_ref, v_ref,
                aw_ref, ab_ref, pw1_ref, pb1_ref, pw2_ref, pb2_ref,
                vw1_ref, vb1_ref, vw2_ref, vb2_ref,
                pol_ref, val_ref, wsum_ref, hmax_ref):
    i = pl.program_id(0)
    h2 = _bn(acc_ref, y2_ref, b_ref, g_ref, be_ref, m_ref, v_ref)
    logits = jnp.sum(h2 * aw_ref[...], axis=1, keepdims=True) + ab_ref[...]
    w = 1.0 / (1.0 + jnp.exp(-logits))
    part_sum = jnp.sum(w * h2, axis=0, keepdims=True)
    part_max = jnp.max(h2, axis=0, keepdims=True)

    @pl.when(i == 0)
    def _():
        wsum_ref[...] = part_sum
        hmax_ref[...] = part_max

    @pl.when(i > 0)
    def _():
        wsum_ref[...] = wsum_ref[...] + part_sum
        hmax_ref[...] = jnp.maximum(hmax_ref[...], part_max)

    @pl.when(i == GRID - 1)
    def _():
        gf = jnp.concatenate([wsum_ref[...], hmax_ref[...]], axis=1)
        p_hidden = jnp.maximum(
            jnp.dot(gf, pw1_ref[...], preferred_element_type=jnp.float32)
            + pb1_ref[...], 0.0)
        pol_ref[...] = (jnp.dot(p_hidden, pw2_ref[...],
                                preferred_element_type=jnp.float32)
                        + pb2_ref[...])
        v_hidden = jnp.maximum(
            jnp.dot(gf, vw1_ref[...], preferred_element_type=jnp.float32)
            + vb1_ref[...], 0.0)
        val_ref[...] = jnp.tanh(
            jnp.sum(v_hidden * vw2_ref[...], axis=1, keepdims=True)
            + vb2_ref[...])


@jax.jit
def _final(acc, y2, b, g, be, m, v, aw_row, ab, pw1, pb1, pw2, pb2,
           vw1, vb1, vw2_row, vb2):
    vec = pl.BlockSpec((1, H), lambda i: (0, 0))
    one = pl.BlockSpec((1, 1), lambda i: (0, 0))
    return pl.pallas_call(
        _final_body,
        grid=(GRID,),
        in_specs=[
            pl.BlockSpec((2, BLK, W), lambda i: (0, i, 0)),
            pl.BlockSpec((BLK, W), lambda i: (i, 0)),
            vec, vec, vec, vec, vec,
            vec, one,
            pl.BlockSpec((W, 128), lambda i: (0, 0)),
            pl.BlockSpec((1, 128), lambda i: (0, 0)),
            pl.BlockSpec((128, 128), lambda i: (0, 0)),
            pl.BlockSpec((1, 128), lambda i: (0, 0)),
            pl.BlockSpec((W, H), lambda i: (0, 0)),
            vec, vec, one,
        ],
        out_specs=[
            pl.BlockSpec((1, 128), lambda i: (0, 0)),
            pl.BlockSpec((1, 1), lambda i: (0, 0)),
        ],
        out_shape=[
            jax.ShapeDtypeStruct((1, 128), jnp.float32),
            jax.ShapeDtypeStruct((1, 1), jnp.float32),
        ],
        scratch_shapes=[
            pltpu.VMEM((1, H), jnp.float32),
            pltpu.VMEM((1, H), jnp.float32),
        ],
    )(acc, y2, b, g, be, m, v, aw_row, ab, pw1, pb1, pw2, pb2,
      vw1, vb1, vw2_row, vb2)


def kernel(x, edge_index, W1, b1, rW1, rb1, g1, be1, m1, v1,
           W2, b2, rW2, rb2, g2, be2, m2, v2, aW, ab,
           pW1, pb1, pW2, pb2, vW1, vb1, vW2, vb2):
    src = edge_index[0]
    dst = edge_index[1]
    pad = E_PAD - E
    pad_iota = jnp.arange(pad, dtype=jnp.int32)
    src_t = jnp.concatenate([src, pad_iota % N]
                            ).reshape(NUM_TILES, K, CHUNK)
    dst_t = jnp.concatenate([dst, N + pad_iota % (ACC_ROWS - N)]
                            ).reshape(NUM_TILES, K, CHUNK)
    zeros_t = jnp.zeros((ACC_ROWS, W), jnp.float32)

    wcat1 = jnp.concatenate([W1, rW1], axis=1)
    wcat2 = jnp.concatenate([W2, rW2], axis=1)
    row = lambda a: a.reshape(1, -1)

    y1 = _proj1(x, wcat1, row(rb1))
    acc1 = _sc_scatter(y1, src_t, dst_t, zeros_t)
    y2 = _epi1(acc1, y1, row(b1), row(g1), row(be1), row(m1), row(v1),
               wcat2, row(rb2))
    acc2 = _sc_scatter(y2, src_t, dst_t, zeros_t)
    pol, val = _final(acc2, y2, row(b2), row(g2), row(be2), row(m2), row(v2),
                      row(aW.T), row(ab), pW1, row(pb1), pW2, row(pb2),
                      vW1, row(vb1), row(vW2.T), row(vb2))
    return (pol, val)


# speedup vs baseline: 1.2403x; 1.2023x over previous
import jax
import jax.numpy as jnp
from jax import lax
from jax.experimental import pallas as pl
from jax.experimental.pallas import tpu as pltpu
from jax.experimental.pallas import tpu_sc as plsc

N = 10000
D = 256
H = 64
W = 2 * H
E = 160000

NUM_TILES = 32
CHUNK = 128
K = 40
E_PAD = NUM_TILES * K * CHUNK
ACC_ROWS = 10240
ZERO_ROWS_PER_TILE = ACC_ROWS // 16

BLK = 1000
GRID = N // BLK


NBUF = 2


def _sc_scatter_body(p_hbm, src_hbm, dst_hbm, zeros_hbm, out_hbm,
                     src_v, dst_v, buf0, buf1,
                     acc, g0, g1):
    c = lax.axis_index("c")
    s = lax.axis_index("s")
    tid = c * 16 + s
    bufs = (buf0, buf1)
    gsem = (g0, g1)

    pltpu.sync_copy(src_hbm.at[tid], src_v)
    pltpu.sync_copy(dst_hbm.at[tid], dst_v)
    for b in range(NBUF):
        pltpu.async_copy(p_hbm.at[src_v.at[b]], bufs[b], gsem[b])

    pltpu.sync_copy(zeros_hbm.at[pl.ds(s * ZERO_ROWS_PER_TILE, ZERO_ROWS_PER_TILE)],
                    acc.at[pl.ds(s * ZERO_ROWS_PER_TILE, ZERO_ROWS_PER_TILE)])
    plsc.subcore_barrier()

    def body(i, _):
        j0 = i * 2
        j1 = j0 + 1
        pltpu.make_async_copy(p_hbm.at[src_v.at[j0]], buf0, g0).wait()
        pltpu.sync_copy(buf0, acc.at[dst_v.at[j0]], add=True)

        @pl.when(j0 + 2 < K)
        def _():
            pltpu.async_copy(p_hbm.at[src_v.at[j0 + 2]], buf0, g0)

        pltpu.make_async_copy(p_hbm.at[src_v.at[j1]], buf1, g1).wait()
        pltpu.sync_copy(buf1, acc.at[dst_v.at[j1]], add=True)

        @pl.when(j1 + 2 < K)
        def _():
            pltpu.async_copy(p_hbm.at[src_v.at[j1 + 2]], buf1, g1)

        return 0

    lax.fori_loop(0, K // 2, body, 0)
    plsc.subcore_barrier()

    pltpu.sync_copy(acc.at[pl.ds(s * ZERO_ROWS_PER_TILE, ZERO_ROWS_PER_TILE)],
                    out_hbm.at[c, pl.ds(s * ZERO_ROWS_PER_TILE, ZERO_ROWS_PER_TILE)])


@jax.jit
def _sc_scatter(p, src_t, dst_t, zeros_t):
    mesh = plsc.VectorSubcoreMesh(core_axis_name="c", subcore_axis_name="s")
    return pl.kernel(
        _sc_scatter_body,
        out_type=jax.ShapeDtypeStruct((2, ACC_ROWS, W), jnp.float32),
        mesh=mesh,
        scratch_types=(
            [pltpu.VMEM((K, CHUNK), jnp.int32)] * 2
            + [pltpu.VMEM((CHUNK, W), jnp.float32)] * NBUF
            + [pltpu.VMEM_SHARED((ACC_ROWS, W), jnp.float32)]
            + [pltpu.SemaphoreType.DMA] * NBUF
        ),
    )(p, src_t, dst_t, zeros_t)


def _proj1_body(x_ref, wcat_ref, rb_ref, y_ref):
    y = jnp.dot(x_ref[...], wcat_ref[...], preferred_element_type=jnp.float32)
    y_ref[...] = jnp.concatenate(
        [y[:, :H], jnp.maximum(y[:, H:] + rb_ref[...], 0.0)], axis=1)


@jax.jit
def _proj1(x, wcat, rb):
    return pl.pallas_call(
        _proj1_body,
        grid=(GRID,),
        in_specs=[
            pl.BlockSpec((BLK, D), lambda i: (i, 0)),
            pl.BlockSpec((D, W), lambda i: (0, 0)),
            pl.BlockSpec((1, H), lambda i: (0, 0)),
        ],
        out_specs=pl.BlockSpec((BLK, W), lambda i: (i, 0)),
        out_shape=jax.ShapeDtypeStruct((N, W), jnp.float32),
    )(x, wcat, rb)


def _bn(acc_ref, y_ref, b_ref, g_ref, be_ref, m_ref, v_ref):
    agg = acc_ref[0, :, :H] + acc_ref[1, :, :H]
    out = jnp.maximum(agg + b_ref[...], 0.0) + y_ref[:, H:]
    scale = g_ref[...] * lax.rsqrt(v_ref[...] + 1e-5)
    return (out - m_ref[...]) * scale + be_ref[...]


def _epi1_body(acc_ref, y1_ref, b_ref, g_ref, be_ref, m_ref, v_ref,
               wcat_ref, rb2_ref, y2_ref):
    h1 = _bn(acc_ref, y1_ref, b_ref, g_ref, be_ref, m_ref, v_ref)
    y = jnp.dot(h1, wcat_ref[...], preferred_element_type=jnp.float32)
    y2_ref[...] = jnp.concatenate(
        [y[:, :H], jnp.maximum(y[:, H:] + rb2_ref[...], 0.0)], axis=1)


@jax.jit
def _epi1(acc, y1, b, g, be, m, v, wcat2, rb2):
    vec = pl.BlockSpec((1, H), lambda i: (0, 0))
    return pl.pallas_call(
        _epi1_body,
        grid=(GRID,),
        in_specs=[
            pl.BlockSpec((2, BLK, W), lambda i: (0, i, 0)),
            pl.BlockSpec((BLK, W), lambda i: (i, 0)),
            vec, vec, vec, vec, vec,
            pl.BlockSpec((H, W), lambda i: (0, 0)),
            vec,
        ],
        out_specs=pl.BlockSpec((BLK, W), lambda i: (i, 0)),
        out_shape=jax.ShapeDtypeStruct((N, W), jnp.float32),
    )(acc, y1, b, g, be, m, v, wcat2, rb2)


def _final_body(acc_ref, y2_ref, b_ref, g_ref, be_ref, m_ref, v_ref,
                aw_ref, ab_ref, pw1_ref, pb1_ref, pw2_ref, pb2_ref,
                vw1_ref, vb1_ref, vw2_ref, vb2_ref,
                pol_ref, val_ref, wsum_ref, hmax_ref):
    i = pl.program_id(0)
    h2 = _bn(acc_ref, y2_ref, b_ref, g_ref, be_ref, m_ref, v_ref)
    logits = jnp.sum(h2 * aw_ref[...], axis=1, keepdims=True) + ab_ref[...]
    w = 1.0 / (1.0 + jnp.exp(-logits))
    part_sum = jnp.sum(w * h2, axis=0, keepdims=True)
    part_max = jnp.max(h2, axis=0, keepdims=True)

    @pl.when(i == 0)
    def _():
        wsum_ref[...] = part_sum
        hmax_ref[...] = part_max

    @pl.when(i > 0)
    def _():
        wsum_ref[...] = wsum_ref[...] + part_sum
        hmax_ref[...] = jnp.maximum(hmax_ref[...], part_max)

    @pl.when(i == GRID - 1)
    def _():
        gf = jnp.concatenate([wsum_ref[...], hmax_ref[...]], axis=1)
        p_hidden = jnp.maximum(
            jnp.dot(gf, pw1_ref[...], preferred_element_type=jnp.float32)
            + pb1_ref[...], 0.0)
        pol_ref[...] = (jnp.dot(p_hidden, pw2_ref[...],
                                preferred_element_type=jnp.float32)
                        + pb2_ref[...])
        v_hidden = jnp.maximum(
            jnp.dot(gf, vw1_ref[...], preferred_element_type=jnp.float32)
            + vb1_ref[...], 0.0)
        val_ref[...] = jnp.tanh(
            jnp.sum(v_hidden * vw2_ref[...], axis=1, keepdims=True)
            + vb2_ref[...])


@jax.jit
def _final(acc, y2, b, g, be, m, v, aw_row, ab, pw1, pb1, pw2, pb2,
           vw1, vb1, vw2_row, vb2):
    vec = pl.BlockSpec((1, H), lambda i: (0, 0))
    one = pl.BlockSpec((1, 1), lambda i: (0, 0))
    return pl.pallas_call(
        _final_body,
        grid=(GRID,),
        in_specs=[
            pl.BlockSpec((2, BLK, W), lambda i: (0, i, 0)),
            pl.BlockSpec((BLK, W), lambda i: (i, 0)),
            vec, vec, vec, vec, vec,
            vec, one,
            pl.BlockSpec((W, 128), lambda i: (0, 0)),
            pl.BlockSpec((1, 128), lambda i: (0, 0)),
            pl.BlockSpec((128, 128), lambda i: (0, 0)),
            pl.BlockSpec((1, 128), lambda i: (0, 0)),
            pl.BlockSpec((W, H), lambda i: (0, 0)),
            vec, vec, one,
        ],
        out_specs=[
            pl.BlockSpec((1, 128), lambda i: (0, 0)),
            pl.BlockSpec((1, 1), lambda i: (0, 0)),
        ],
        out_shape=[
            jax.ShapeDtypeStruct((1, 128), jnp.float32),
            jax.ShapeDtypeStruct((1, 1), jnp.float32),
        ],
        scratch_shapes=[
            pltpu.VMEM((1, H), jnp.float32),
            pltpu.VMEM((1, H), jnp.float32),
        ],
    )(acc, y2, b, g, be, m, v, aw_row, ab, pw1, pb1, pw2, pb2,
      vw1, vb1, vw2_row, vb2)


def kernel(x, edge_index, W1, b1, rW1, rb1, g1, be1, m1, v1,
           W2, b2, rW2, rb2, g2, be2, m2, v2, aW, ab,
           pW1, pb1, pW2, pb2, vW1, vb1, vW2, vb2):
    src = edge_index[0]
    dst = edge_index[1]
    pad = E_PAD - E
    pad_iota = jnp.arange(pad, dtype=jnp.int32)
    src_t = jnp.concatenate([src, pad_iota % N]
                            ).reshape(NUM_TILES, K, CHUNK)
    dst_t = jnp.concatenate([dst, N + pad_iota % (ACC_ROWS - N)]
                            ).reshape(NUM_TILES, K, CHUNK)
    zeros_t = jnp.zeros((ACC_ROWS, W), jnp.float32)

    wcat1 = jnp.concatenate([W1, rW1], axis=1)
    wcat2 = jnp.concatenate([W2, rW2], axis=1)
    row = lambda a: a.reshape(1, -1)

    y1 = _proj1(x, wcat1, row(rb1))
    acc1 = _sc_scatter(y1, src_t, dst_t, zeros_t)
    y2 = _epi1(acc1, y1, row(b1), row(g1), row(be1), row(m1), row(v1),
               wcat2, row(rb2))
    acc2 = _sc_scatter(y2, src_t, dst_t, zeros_t)
    pol, val = _final(acc2, y2, row(b2), row(g2), row(be2), row(m2), row(v2),
                      row(aW.T), row(ab), pW1, row(pb1), pW2, row(pb2),
                      vW1, row(vb1), row(vW2.T), row(vb2))
    return (pol, val)
